# baseline (speedup 1.0000x reference)
import functools

import jax
import jax.numpy as jnp
from jax import lax
from jax.experimental import pallas as pl
from jax.experimental.pallas import tpu as pltpu
from jax.experimental.pallas import tpu_sc as plsc

D4 = 256
D8 = 128
VOCAB_ROWS = 457
DOUT = D4 + D4 + D8
NC, NS = 2, 16
NW = NC * NS


def _sc_gather_into(table, ids, out_ref, B):
    b_per_w = B // NW
    CH = 128
    n_ch = b_per_w // CH
    mesh = plsc.VectorSubcoreMesh(core_axis_name="c", subcore_axis_name="s")

    @functools.partial(
        pl.kernel,
        out_type=(),
        mesh=mesh,
        scratch_types=[
            pltpu.VMEM((b_per_w,), jnp.int32),
            pltpu.VMEM((CH, D4), jnp.float32),
            pltpu.VMEM((CH, D4), jnp.float32),
            pltpu.VMEM((CH, D4), jnp.float32),
            pltpu.SemaphoreType.DMA,
            pltpu.SemaphoreType.DMA,
            pltpu.SemaphoreType.DMA,
            pltpu.SemaphoreType.DMA,
            pltpu.SemaphoreType.DMA,
            pltpu.SemaphoreType.DMA,
        ],
    )
    def k(table_hbm, idx_hbm, out_hbm, idx_v, buf0, buf1, buf2,
          rsem0, rsem1, rsem2, wsem0, wsem1, wsem2):
        wid = lax.axis_index("s") * NC + lax.axis_index("c")
        base = wid * b_per_w
        pltpu.sync_copy(idx_hbm.at[pl.ds(base, b_per_w)], idx_v)
        bufs = (buf0, buf1, buf2)
        rsems = (rsem0, rsem1, rsem2)
        wsems = (wsem0, wsem1, wsem2)
        hr = [None] * n_ch
        hw = [None] * n_ch
        for p in range(min(2, n_ch)):
            hr[p] = pltpu.async_copy(
                table_hbm.at[idx_v.at[pl.ds(p * CH, CH)]], bufs[p], rsems[p])
        for c in range(n_ch):
            hr[c].wait()
            hw[c] = pltpu.async_copy(
                bufs[c % 3],
                out_hbm.at[pl.ds(base + c * CH, CH), pl.ds(0, D4)],
                wsems[c % 3])
            if c + 2 < n_ch:
                if c - 1 >= 0:
                    hw[c - 1].wait()
                hr[c + 2] = pltpu.async_copy(
                    table_hbm.at[idx_v.at[pl.ds((c + 2) * CH, CH)]],
                    bufs[(c + 2) % 3], rsems[(c + 2) % 3])
        for c in range(max(0, n_ch - 3), n_ch):
            hw[c].wait()

    return k(table, ids, out_ref)


def _tc_mlps(X8, M1, W2, b2, M3, W4, b4):
    B = X8.shape[0]
    bs = 2048
    grid = (B // bs, 2)
    dn = (((1,), (0,)), ((), ()))
    dnT = (((1,), (1,)), ((), ()))

    def body(X8_ref, M1_ref, W2_ref, b2_ref, M3_ref, W4_ref, b4_ref,
             out_ref):
        j = pl.program_id(1)
        x = X8_ref[:, :]

        @pl.when(j == 0)
        def _():
            h = jnp.maximum(lax.dot_general(
                x, M1_ref[:, :], dn, preferred_element_type=jnp.float32), 0.0)
            out_ref[:, :] = lax.dot_general(
                h.astype(jnp.bfloat16), W2_ref[:, :], dnT,
                preferred_element_type=jnp.float32) + b2_ref[:][None, :]

        @pl.when(j == 1)
        def _():
            g = jnp.maximum(lax.dot_general(
                x, M3_ref[:, :], dn, preferred_element_type=jnp.float32), 0.0)
            out_ref[:, :D8] = jax.nn.sigmoid(lax.dot_general(
                g.astype(jnp.bfloat16), W4_ref[:, :], dnT,
                preferred_element_type=jnp.float32) + b4_ref[:][None, :])

    return pl.pallas_call(
        body,
        grid=grid,
        in_specs=[
            pl.BlockSpec((bs, 8), lambda i, j: (i, 0)),
            pl.BlockSpec((8, D4), lambda i, j: (0, 0)),
            pl.BlockSpec((D4, D4), lambda i, j: (0, 0)),
            pl.BlockSpec((D4,), lambda i, j: (0,)),
            pl.BlockSpec((8, D8), lambda i, j: (0, 0)),
            pl.BlockSpec((D8, D8), lambda i, j: (0, 0)),
            pl.BlockSpec((D8,), lambda i, j: (0,)),
        ],
        out_specs=pl.BlockSpec((bs, D4), lambda i, j: (i, j + 1)),
        out_shape=jax.ShapeDtypeStruct((B, DOUT), jnp.float32),
    )(X8, M1, W2, b2, M3, W4, b4)


def kernel(time_control_id, white_remaining, black_remaining, table,
           W1, b1, W2, b2, W3, b3, W4, b4):
    B = time_control_id.shape[0]
    ids = time_control_id.astype(jnp.int32)
    one = jnp.ones((B, 1), jnp.float32)
    X8 = jnp.concatenate(
        [white_remaining[:, None], black_remaining[:, None], one,
         jnp.zeros((B, 5), jnp.float32)], axis=1)
    zpad = jnp.zeros((5, D4), jnp.float32)
    M1 = jnp.concatenate([W1[:, 0][None], W1[:, 1][None], b1[None], zpad], 0)
    M3 = jnp.concatenate(
        [W3[:, 0][None], W3[:, 1][None], b3[None], zpad[:, :D8]], 0)
    buf = _tc_mlps(X8, M1, W2.astype(jnp.bfloat16), b2,
                   M3, W4.astype(jnp.bfloat16), b4)
    ref = jax.new_ref(buf)
    _sc_gather_into(table, ids, ref, B)
    return jax.freeze(ref)


# speedup vs baseline: 1.1103x; 1.1103x over previous
import functools

import jax
import jax.numpy as jnp
from jax import lax
from jax.experimental import pallas as pl
from jax.experimental.pallas import tpu as pltpu
from jax.experimental.pallas import tpu_sc as plsc

D4 = 256
D8 = 128
VOCAB_ROWS = 457
DOUT = D4 + D4 + D8
NC, NS = 2, 16
NW = NC * NS


def _sc_gather_into(table, ids, B):
    b_per_w = B // NW
    CH = 64
    NBUF = 7
    DEPTH = 4
    n_ch = b_per_w // CH
    mesh = plsc.VectorSubcoreMesh(core_axis_name="c", subcore_axis_name="s")

    @functools.partial(
        pl.kernel,
        out_type=jax.ShapeDtypeStruct((B, DOUT), jnp.float32),
        mesh=mesh,
        scratch_types=(
            [pltpu.VMEM((b_per_w,), jnp.int32)]
            + [pltpu.VMEM((CH, D4), jnp.float32)] * NBUF
            + [pltpu.SemaphoreType.DMA] * (2 * NBUF)
        ),
    )
    def k(table_hbm, idx_hbm, out_hbm, idx_v, *rest):
        bufs = rest[:NBUF]
        rsems = rest[NBUF:2 * NBUF]
        wsems = rest[2 * NBUF:]
        wid = lax.axis_index("s") * NC + lax.axis_index("c")
        base = wid * b_per_w
        pltpu.sync_copy(idx_hbm.at[pl.ds(base, b_per_w)], idx_v)
        hr = [None] * n_ch
        hw = [None] * n_ch
        for p in range(min(DEPTH, n_ch)):
            hr[p] = pltpu.async_copy(
                table_hbm.at[idx_v.at[pl.ds(p * CH, CH)]],
                bufs[p % NBUF], rsems[p % NBUF])
        for c in range(n_ch):
            hr[c].wait()
            hw[c] = pltpu.async_copy(
                bufs[c % NBUF],
                out_hbm.at[pl.ds(base + c * CH, CH), pl.ds(0, D4)],
                wsems[c % NBUF])
            nxt = c + DEPTH
            if nxt < n_ch:
                if nxt - NBUF >= 0:
                    hw[nxt - NBUF].wait()
                hr[nxt] = pltpu.async_copy(
                    table_hbm.at[idx_v.at[pl.ds(nxt * CH, CH)]],
                    bufs[nxt % NBUF], rsems[nxt % NBUF])
        for c in range(max(0, n_ch - NBUF), n_ch):
            hw[c].wait()

    return k(table, ids)


def _tc_mlps_inplace(buf, X8, M1, W2, b2, M3, W4, b4):
    B = X8.shape[0]
    bs = 2048
    grid = (B // bs, 2)
    dn = (((1,), (0,)), ((), ()))
    dnT = (((1,), (1,)), ((), ()))

    def body(buf_ref, X8_ref, M1_ref, W2_ref, b2_ref, M3_ref, W4_ref, b4_ref,
             out_ref):
        del buf_ref
        j = pl.program_id(1)
        x = X8_ref[:, :]

        @pl.when(j == 0)
        def _():
            h = jnp.maximum(lax.dot_general(
                x, M1_ref[:, :], dn, preferred_element_type=jnp.float32), 0.0)
            out_ref[:, :] = lax.dot_general(
                h.astype(jnp.bfloat16), W2_ref[:, :], dnT,
                preferred_element_type=jnp.float32) + b2_ref[:][None, :]

        @pl.when(j == 1)
        def _():
            g = jnp.maximum(lax.dot_general(
                x, M3_ref[:, :], dn, preferred_element_type=jnp.float32), 0.0)
            out_ref[:, :D8] = jax.nn.sigmoid(lax.dot_general(
                g.astype(jnp.bfloat16), W4_ref[:, :], dnT,
                preferred_element_type=jnp.float32) + b4_ref[:][None, :])

    return pl.pallas_call(
        body,
        grid=grid,
        in_specs=[
            pl.BlockSpec(memory_space=pl.ANY),
            pl.BlockSpec((bs, 8), lambda i, j: (i, 0)),
            pl.BlockSpec((8, D4), lambda i, j: (0, 0)),
            pl.BlockSpec((D4, D4), lambda i, j: (0, 0)),
            pl.BlockSpec((D4,), lambda i, j: (0,)),
            pl.BlockSpec((8, D8), lambda i, j: (0, 0)),
            pl.BlockSpec((D8, D8), lambda i, j: (0, 0)),
            pl.BlockSpec((D8,), lambda i, j: (0,)),
        ],
        out_specs=pl.BlockSpec((bs, D4), lambda i, j: (i, j + 1)),
        out_shape=jax.ShapeDtypeStruct((B, DOUT), jnp.float32),
        input_output_aliases={0: 0},
    )(buf, X8, M1, W2, b2, M3, W4, b4)


def kernel(time_control_id, white_remaining, black_remaining, table,
           W1, b1, W2, b2, W3, b3, W4, b4):
    B = time_control_id.shape[0]
    ids = time_control_id.astype(jnp.int32)
    one = jnp.ones((B, 1), jnp.float32)
    X8 = jnp.concatenate(
        [white_remaining[:, None], black_remaining[:, None], one,
         jnp.zeros((B, 5), jnp.float32)], axis=1)
    zpad = jnp.zeros((5, D4), jnp.float32)
    M1 = jnp.concatenate([W1[:, 0][None], W1[:, 1][None], b1[None], zpad], 0)
    M3 = jnp.concatenate(
        [W3[:, 0][None], W3[:, 1][None], b3[None], zpad[:, :D8]], 0)
    buf = _sc_gather_into(table, ids, B)
    return _tc_mlps_inplace(buf, X8, M1, W2.astype(jnp.bfloat16), b2,
                            M3, W4.astype(jnp.bfloat16), b4)


# speedup vs baseline: 1.1120x; 1.0015x over previous
import functools

import jax
import jax.numpy as jnp
from jax import lax
from jax.experimental import pallas as pl
from jax.experimental.pallas import tpu as pltpu
from jax.experimental.pallas import tpu_sc as plsc

D4 = 256
D8 = 128
VOCAB_ROWS = 457
DOUT = D4 + D4 + D8
NC, NS = 2, 16
NW = NC * NS


def _sc_gather_into(table, ids, B):
    b_per_w = B // NW
    CH = 128
    NBUF = 3
    DEPTH = 2
    n_ch = b_per_w // CH
    mesh = plsc.VectorSubcoreMesh(core_axis_name="c", subcore_axis_name="s")

    @functools.partial(
        pl.kernel,
        out_type=jax.ShapeDtypeStruct((B, DOUT), jnp.float32),
        mesh=mesh,
        scratch_types=(
            [pltpu.VMEM((b_per_w,), jnp.int32)]
            + [pltpu.VMEM((CH, D4), jnp.float32)] * NBUF
            + [pltpu.SemaphoreType.DMA] * (2 * NBUF)
        ),
    )
    def k(table_hbm, idx_hbm, out_hbm, idx_v, *rest):
        bufs = rest[:NBUF]
        rsems = rest[NBUF:2 * NBUF]
        wsems = rest[2 * NBUF:]
        wid = lax.axis_index("s") * NC + lax.axis_index("c")
        base = wid * b_per_w
        pltpu.sync_copy(idx_hbm.at[pl.ds(base, b_per_w)], idx_v)
        hr = [None] * n_ch
        hw = [None] * n_ch
        for p in range(min(DEPTH, n_ch)):
            hr[p] = pltpu.async_copy(
                table_hbm.at[idx_v.at[pl.ds(p * CH, CH)]],
                bufs[p % NBUF], rsems[p % NBUF])
        for c in range(n_ch):
            hr[c].wait()
            hw[c] = pltpu.async_copy(
                bufs[c % NBUF],
                out_hbm.at[pl.ds(base + c * CH, CH), pl.ds(0, D4)],
                wsems[c % NBUF])
            nxt = c + DEPTH
            if nxt < n_ch:
                if nxt - NBUF >= 0:
                    hw[nxt - NBUF].wait()
                hr[nxt] = pltpu.async_copy(
                    table_hbm.at[idx_v.at[pl.ds(nxt * CH, CH)]],
                    bufs[nxt % NBUF], rsems[nxt % NBUF])
        for c in range(max(0, n_ch - NBUF), n_ch):
            hw[c].wait()

    return k(table, ids)


def _tc_mlps_inplace(buf, X8, M1, W2, b2, M3, W4, b4):
    B = X8.shape[0]
    bs = 2048
    grid = (B // bs, 2)
    dn = (((1,), (0,)), ((), ()))
    dnT = (((1,), (1,)), ((), ()))

    def body(buf_ref, X8_ref, M1_ref, W2_ref, b2_ref, M3_ref, W4_ref, b4_ref,
             out_ref):
        del buf_ref
        j = pl.program_id(1)
        x = X8_ref[:, :]

        @pl.when(j == 0)
        def _():
            h = jnp.maximum(lax.dot_general(
                x, M1_ref[:, :], dn, preferred_element_type=jnp.float32), 0.0)
            out_ref[:, :] = lax.dot_general(
                h.astype(jnp.bfloat16), W2_ref[:, :], dnT,
                preferred_element_type=jnp.float32) + b2_ref[:][None, :]

        @pl.when(j == 1)
        def _():
            g = jnp.maximum(lax.dot_general(
                x, M3_ref[:, :], dn, preferred_element_type=jnp.float32), 0.0)
            out_ref[:, :D8] = jax.nn.sigmoid(lax.dot_general(
                g.astype(jnp.bfloat16), W4_ref[:, :], dnT,
                preferred_element_type=jnp.float32) + b4_ref[:][None, :])

    return pl.pallas_call(
        body,
        grid=grid,
        in_specs=[
            pl.BlockSpec(memory_space=pl.ANY),
            pl.BlockSpec((bs, 8), lambda i, j: (i, 0)),
            pl.BlockSpec((8, D4), lambda i, j: (0, 0)),
            pl.BlockSpec((D4, D4), lambda i, j: (0, 0)),
            pl.BlockSpec((D4,), lambda i, j: (0,)),
            pl.BlockSpec((8, D8), lambda i, j: (0, 0)),
            pl.BlockSpec((D8, D8), lambda i, j: (0, 0)),
            pl.BlockSpec((D8,), lambda i, j: (0,)),
        ],
        out_specs=pl.BlockSpec((bs, D4), lambda i, j: (i, j + 1)),
        out_shape=jax.ShapeDtypeStruct((B, DOUT), jnp.float32),
        input_output_aliases={0: 0},
    )(buf, X8, M1, W2, b2, M3, W4, b4)


def kernel(time_control_id, white_remaining, black_remaining, table,
           W1, b1, W2, b2, W3, b3, W4, b4):
    B = time_control_id.shape[0]
    ids = time_control_id.astype(jnp.int32)
    one = jnp.ones((B, 1), jnp.float32)
    X8 = jnp.concatenate(
        [white_remaining[:, None], black_remaining[:, None], one,
         jnp.zeros((B, 5), jnp.float32)], axis=1)
    zpad = jnp.zeros((5, D4), jnp.float32)
    M1 = jnp.concatenate([W1[:, 0][None], W1[:, 1][None], b1[None], zpad], 0)
    M3 = jnp.concatenate(
        [W3[:, 0][None], W3[:, 1][None], b3[None], zpad[:, :D8]], 0)
    buf = _sc_gather_into(table, ids, B)
    return _tc_mlps_inplace(buf, X8, M1, W2.astype(jnp.bfloat16), b2,
                            M3, W4.astype(jnp.bfloat16), b4)


# speedup vs baseline: 1.3506x; 1.2145x over previous
import functools

import jax
import jax.numpy as jnp
from jax import lax
from jax.experimental import pallas as pl
from jax.experimental.pallas import tpu as pltpu
from jax.experimental.pallas import tpu_sc as plsc

D4 = 256
D8 = 128
VOCAB_ROWS = 457
DOUT = D4 + D4 + D8
NC, NS = 2, 16
NW = NC * NS


def _sc_gather_into(table, ids, B):
    b_per_w = B // NW
    CH = 128
    NBUF = 3
    DEPTH = 2
    n_ch = b_per_w // CH
    mesh = plsc.VectorSubcoreMesh(core_axis_name="c", subcore_axis_name="s")

    @functools.partial(
        pl.kernel,
        out_type=jax.ShapeDtypeStruct((B, DOUT), jnp.float32),
        mesh=mesh,
        scratch_types=(
            [pltpu.VMEM((b_per_w,), jnp.int32)]
            + [pltpu.VMEM((CH, D4), jnp.float32)] * NBUF
            + [pltpu.SemaphoreType.DMA] * (2 * NBUF)
        ),
    )
    def k(table_hbm, idx_hbm, out_hbm, idx_v, *rest):
        bufs = rest[:NBUF]
        rsems = rest[NBUF:2 * NBUF]
        wsems = rest[2 * NBUF:]
        wid = lax.axis_index("s") * NC + lax.axis_index("c")
        base = wid * b_per_w
        pltpu.sync_copy(idx_hbm.at[pl.ds(base, b_per_w)], idx_v)
        hr = [None] * n_ch
        hw = [None] * n_ch
        for p in range(min(DEPTH, n_ch)):
            hr[p] = pltpu.async_copy(
                table_hbm.at[idx_v.at[pl.ds(p * CH, CH)]],
                bufs[p % NBUF], rsems[p % NBUF])
        for c in range(n_ch):
            hr[c].wait()
            hw[c] = pltpu.async_copy(
                bufs[c % NBUF],
                out_hbm.at[pl.ds(base + c * CH, CH), pl.ds(0, D4)],
                wsems[c % NBUF])
            nxt = c + DEPTH
            if nxt < n_ch:
                if nxt - NBUF >= 0:
                    hw[nxt - NBUF].wait()
                hr[nxt] = pltpu.async_copy(
                    table_hbm.at[idx_v.at[pl.ds(nxt * CH, CH)]],
                    bufs[nxt % NBUF], rsems[nxt % NBUF])
        for c in range(max(0, n_ch - NBUF), n_ch):
            hw[c].wait()

    return k(table, ids)


def _tc_mlps_inplace(buf, X8, M1, W2, b2, M3, W4, b4):
    B = X8.shape[0]
    bs = 8192
    grid = (B // bs, 2)
    dn = (((1,), (0,)), ((), ()))
    dnT = (((1,), (1,)), ((), ()))

    def body(buf_ref, X8_ref, M1_ref, W2_ref, b2_ref, M3_ref, W4_ref, b4_ref,
             out_ref):
        del buf_ref
        j = pl.program_id(1)
        x = X8_ref[:, :]

        @pl.when(j == 0)
        def _():
            h = jnp.maximum(lax.dot_general(
                x, M1_ref[:, :], dn, preferred_element_type=jnp.float32), 0.0)
            out_ref[:, :] = lax.dot_general(
                h.astype(jnp.bfloat16), W2_ref[:, :], dnT,
                preferred_element_type=jnp.float32) + b2_ref[:][None, :]

        @pl.when(j == 1)
        def _():
            g = jnp.maximum(lax.dot_general(
                x, M3_ref[:, :], dn, preferred_element_type=jnp.float32), 0.0)
            out_ref[:, :D8] = jax.nn.sigmoid(lax.dot_general(
                g.astype(jnp.bfloat16), W4_ref[:, :], dnT,
                preferred_element_type=jnp.float32) + b4_ref[:][None, :])

    return pl.pallas_call(
        body,
        grid=grid,
        in_specs=[
            pl.BlockSpec(memory_space=pl.ANY),
            pl.BlockSpec((bs, 8), lambda i, j: (i, 0)),
            pl.BlockSpec((8, D4), lambda i, j: (0, 0)),
            pl.BlockSpec((D4, D4), lambda i, j: (0, 0)),
            pl.BlockSpec((D4,), lambda i, j: (0,)),
            pl.BlockSpec((8, D8), lambda i, j: (0, 0)),
            pl.BlockSpec((D8, D8), lambda i, j: (0, 0)),
            pl.BlockSpec((D8,), lambda i, j: (0,)),
        ],
        out_specs=pl.BlockSpec((bs, D4), lambda i, j: (i, j + 1)),
        out_shape=jax.ShapeDtypeStruct((B, DOUT), jnp.float32),
        input_output_aliases={0: 0},
    )(buf, X8, M1, W2, b2, M3, W4, b4)


def kernel(time_control_id, white_remaining, black_remaining, table,
           W1, b1, W2, b2, W3, b3, W4, b4):
    B = time_control_id.shape[0]
    ids = time_control_id.astype(jnp.int32)
    one = jnp.ones((B, 1), jnp.float32)
    X8 = jnp.concatenate(
        [white_remaining[:, None], black_remaining[:, None], one,
         jnp.zeros((B, 5), jnp.float32)], axis=1)
    zpad = jnp.zeros((5, D4), jnp.float32)
    M1 = jnp.concatenate([W1[:, 0][None], W1[:, 1][None], b1[None], zpad], 0)
    M3 = jnp.concatenate(
        [W3[:, 0][None], W3[:, 1][None], b3[None], zpad[:, :D8]], 0)
    buf = _sc_gather_into(table, ids, B)
    return _tc_mlps_inplace(buf, X8, M1, W2.astype(jnp.bfloat16), b2,
                            M3, W4.astype(jnp.bfloat16), b4)
